# baseline (speedup 1.0000x reference)
import dataclasses
import functools

import jax
import jax.numpy as jnp
from jax import lax
from jax.experimental import pallas as pl
from jax.experimental.pallas import tpu as pltpu
from jax.experimental.pallas import tpu_sc as plsc

NC = 2
NS = 16
NW = NC * NS
LANES = 16
K = 128
KC = 96
RING = 4
KC64 = 80
RING64 = 5

_MESH = plsc.VectorSubcoreMesh(
    core_axis_name="c", subcore_axis_name="s", num_cores=NC, num_subcores=NS
)

_SC_PARAMS = pltpu.CompilerParams()
if "needs_layout_passes" in pltpu.CompilerParams.__dataclass_fields__:
    _SC_PARAMS = dataclasses.replace(_SC_PARAMS, needs_layout_passes=False)


def _wid():
    return lax.axis_index("s") * NC + lax.axis_index("c")


def _hist(eflat, n_nodes):
    e_pad = eflat.shape[0] // 2
    assert e_pad % NW == 0
    epw = e_pad // NW
    assert epw % LANES == 0

    @functools.partial(
        pl.kernel,
        out_type=jax.ShapeDtypeStruct((NW, n_nodes), jnp.float32),
        mesh=_MESH,
        scratch_types=[
            pltpu.VMEM((n_nodes,), jnp.float32),
            pltpu.VMEM((epw,), jnp.int32),
            pltpu.SemaphoreType.DMA,
        ],
        compiler_params=_SC_PARAMS,
    )
    def hist_kernel(ei_hbm, out_hbm, hist_v, idx_v, sem):
        wid = _wid()
        zero16 = jnp.zeros((LANES,), jnp.float32)
        ones16 = jnp.ones((LANES,), jnp.float32)

        cp = pltpu.async_copy(ei_hbm.at[pl.ds(e_pad + wid * epw, epw)],
                              idx_v, sem)

        @pl.loop(0, n_nodes, step=LANES)
        def _(i):
            hist_v[pl.ds(i, LANES)] = zero16

        cp.wait()

        full = epw // K * K

        @pl.loop(0, full, step=K)
        def _(i):
            for j in range(K // LANES):
                idx16 = idx_v[pl.ds(i + j * LANES, LANES)]
                plsc.addupdate_scatter(hist_v, [idx16], ones16)

        for j in range((epw - full) // LANES):
            idx16 = idx_v[pl.ds(full + j * LANES, LANES)]
            plsc.addupdate_scatter(hist_v, [idx16], ones16)

        pltpu.async_copy(hist_v, out_hbm.at[wid], sem).wait()

    return hist_kernel(eflat)


def _aggregate(feat, eflat):
    n, w = feat.shape
    e_pad = eflat.shape[0] // 2
    assert e_pad % NW == 0
    epw = e_pad // NW
    assert epw % 8 == 0
    nfull = epw // KC
    tail_len = epw - nfull * KC
    NB = RING
    LA = NB - 2
    assert nfull % NB == 0 and nfull > NB, (nfull, NB)
    rps = (n // NS) // 8 * 8
    rtail = n - rps * NS
    assert rtail % 8 == 0

    @functools.partial(
        pl.kernel,
        out_type=jax.ShapeDtypeStruct((NC, n, w), jnp.float32),
        mesh=_MESH,
        scratch_types=[
            pltpu.VMEM_SHARED((n, w), jnp.float32),
            pltpu.VMEM((NB, KC), jnp.int32),
            pltpu.VMEM((NB, KC), jnp.int32),
            pltpu.VMEM((NB, KC, w), jnp.float32),
            pltpu.VMEM((1, max(tail_len, 8)), jnp.int32),
        ] + [pltpu.SemaphoreType.DMA] * (1 + 4 * NB),
    )
    def agg_kernel(feat_hbm, ei_hbm, part_hbm,
                   acc_sh, sidx, didx, rows, dtail, semb, *sems_flat):
        cid = lax.axis_index("c")
        sid = lax.axis_index("s")
        wid = _wid()
        base = sid * rps
        ebase = wid * epw
        semx = sems_flat[0:NB]
        semd = sems_flat[NB:2 * NB]
        semg = sems_flat[2 * NB:3 * NB]
        sems = sems_flat[3 * NB:4 * NB]

        zero16 = jnp.zeros((LANES,), jnp.float32)

        @pl.loop(0, KC)
        def _(r):
            for j in range(w // LANES):
                rows[0, r, pl.ds(j * LANES, LANES)] = zero16

        nz = rps // KC
        zrem = rps - nz * KC

        @pl.loop(0, nz)
        def _(k):
            zoff = pl.multiple_of(base + k * KC, 8)
            pltpu.async_copy(rows.at[0], acc_sh.at[pl.ds(zoff, KC)],
                             semb).wait()
        if zrem:
            pltpu.async_copy(rows.at[0].at[pl.ds(0, zrem)],
                             acc_sh.at[pl.ds(base + nz * KC, zrem)],
                             semb).wait()
        if rtail:
            @pl.when(sid == NS - 1)
            def _():
                pltpu.async_copy(rows.at[0].at[pl.ds(0, rtail)],
                                 acc_sh.at[pl.ds(rps * NS, rtail)],
                                 semb).wait()

        plsc.subcore_barrier()

        def _span(chunk, half):
            return pl.ds(half * e_pad + ebase + chunk * KC, KC)

        def sidx_start(chunk, b):
            pltpu.async_copy(ei_hbm.at[_span(chunk, 0)], sidx.at[b], semx[b])

        def sidx_wait(chunk, b):
            pltpu.make_async_copy(ei_hbm.at[_span(chunk, 0)], sidx.at[b],
                                  semx[b]).wait()

        def didx_start(chunk, b):
            pltpu.async_copy(ei_hbm.at[_span(chunk, 1)], didx.at[b], semd[b])

        def didx_wait(chunk, b):
            pltpu.make_async_copy(ei_hbm.at[_span(chunk, 1)], didx.at[b],
                                  semd[b]).wait()

        def gather_start(b):
            pltpu.async_copy(feat_hbm.at[sidx.at[b]], rows.at[b], semg[b])

        def gather_wait(b):
            pltpu.make_async_copy(feat_hbm.at[sidx.at[b]], rows.at[b],
                                  semg[b]).wait()

        def _rows_src(b, nr):
            return rows.at[b] if nr == KC else rows.at[b].at[pl.ds(0, nr)]

        def scatter_start(b):
            pltpu.async_copy(_rows_src(b, KC), acc_sh.at[didx.at[b]],
                             sems[b], add=True)

        def scatter_wait(b):
            pltpu.make_async_copy(_rows_src(b, KC), acc_sh.at[didx.at[b]],
                                  sems[b]).wait()

        for c0 in range(LA + 1):
            sidx_start(c0, c0)
            didx_start(c0, c0)
        for c0 in range(LA):
            sidx_wait(c0, c0)
            gather_start(c0)

        @pl.loop(0, nfull, step=NB)
        def _(c):
            for b in range(NB):
                chunk = c + b
                bg = (b + LA) % NB
                bp = (b + NB - 1) % NB

                @pl.when(chunk + LA < nfull)
                def _():
                    sidx_wait(chunk + LA, bg)
                    gather_start(bg)

                gather_wait(b)
                didx_wait(chunk, b)
                scatter_start(b)

                @pl.when(chunk >= 1)
                def _():
                    scatter_wait(bp)

                @pl.when(chunk + LA + 1 < nfull)
                def _():
                    sidx_start(chunk + LA + 1, bp)
                    didx_start(chunk + LA + 1, bp)

        scatter_wait((nfull - 1) % NB)

        if tail_len:
            toff = nfull * KC
            pltpu.async_copy(ei_hbm.at[pl.ds(e_pad + ebase + toff, tail_len)],
                             dtail.at[0], semd[0]).wait()
            pltpu.async_copy(ei_hbm.at[pl.ds(ebase + toff, tail_len)],
                             sidx.at[0].at[pl.ds(0, tail_len)], semx[0]).wait()
            pltpu.async_copy(
                feat_hbm.at[sidx.at[0].at[pl.ds(0, tail_len)]],
                rows.at[0].at[pl.ds(0, tail_len)], semg[0]).wait()
            pltpu.async_copy(_rows_src(0, tail_len),
                             acc_sh.at[dtail.at[0]], sems[0], add=True).wait()

        plsc.subcore_barrier()

        pltpu.async_copy(acc_sh.at[pl.ds(base, rps)],
                         part_hbm.at[cid, pl.ds(base, rps)], semb).wait()
        if rtail:
            @pl.when(sid == NS - 1)
            def _():
                pltpu.async_copy(acc_sh.at[pl.ds(rps * NS, rtail)],
                                 part_hbm.at[cid, pl.ds(rps * NS, rtail)],
                                 semb).wait()

    return agg_kernel(feat, eflat)


def _aggregate64(feat, eflat):
    KC = KC64
    n, w = feat.shape
    e_pad = eflat.shape[0] // 2
    assert e_pad % NW == 0
    epw = e_pad // NW
    assert epw % 8 == 0
    nfull = epw // KC
    tail_len = epw - nfull * KC
    NB = RING64
    LA = NB - 2
    assert nfull % NB == 0 and nfull > NB, (nfull, NB)
    rps = (n // NS) // 8 * 8
    rtail = n - rps * NS
    assert rtail % 8 == 0

    @functools.partial(
        pl.kernel,
        out_type=jax.ShapeDtypeStruct((NC, n, w), jnp.float32),
        mesh=_MESH,
        scratch_types=[
            pltpu.VMEM_SHARED((n, w), jnp.float32),
            pltpu.VMEM((NB, KC), jnp.int32),
            pltpu.VMEM((NB, KC), jnp.int32),
            pltpu.VMEM((NB, KC, w), jnp.float32),
            pltpu.VMEM((1, max(tail_len, 8)), jnp.int32),
        ] + [pltpu.SemaphoreType.DMA] * (1 + 4 * NB),
        compiler_params=dataclasses.replace(
            _SC_PARAMS, use_tc_tiling_on_sc=False),
    )
    def agg_kernel(feat_hbm, ei_hbm, part_hbm,
                   acc_sh, sidx, didx, rows, dtail, semb, *sems_flat):
        cid = lax.axis_index("c")
        sid = lax.axis_index("s")
        wid = _wid()
        base = sid * rps
        ebase = wid * epw
        semx = sems_flat[0:NB]
        semd = sems_flat[NB:2 * NB]
        semg = sems_flat[2 * NB:3 * NB]
        sems = sems_flat[3 * NB:4 * NB]

        zero16 = jnp.zeros((LANES,), jnp.float32)

        @pl.loop(0, KC)
        def _(r):
            for j in range(w // LANES):
                rows[0, r, pl.ds(j * LANES, LANES)] = zero16

        nz = rps // KC
        zrem = rps - nz * KC

        @pl.loop(0, nz)
        def _(k):
            zoff = pl.multiple_of(base + k * KC, 8)
            pltpu.async_copy(rows.at[0], acc_sh.at[pl.ds(zoff, KC)],
                             semb).wait()
        if zrem:
            pltpu.async_copy(rows.at[0].at[pl.ds(0, zrem)],
                             acc_sh.at[pl.ds(base + nz * KC, zrem)],
                             semb).wait()
        if rtail:
            @pl.when(sid == NS - 1)
            def _():
                pltpu.async_copy(rows.at[0].at[pl.ds(0, rtail)],
                                 acc_sh.at[pl.ds(rps * NS, rtail)],
                                 semb).wait()

        plsc.subcore_barrier()

        def _span(chunk, half):
            return pl.ds(half * e_pad + ebase + chunk * KC, KC)

        def sidx_start(chunk, b):
            pltpu.async_copy(ei_hbm.at[_span(chunk, 0)], sidx.at[b], semx[b])

        def sidx_wait(chunk, b):
            pltpu.make_async_copy(ei_hbm.at[_span(chunk, 0)], sidx.at[b],
                                  semx[b]).wait()

        def didx_start(chunk, b):
            pltpu.async_copy(ei_hbm.at[_span(chunk, 1)], didx.at[b], semd[b])

        def didx_wait(chunk, b):
            pltpu.make_async_copy(ei_hbm.at[_span(chunk, 1)], didx.at[b],
                                  semd[b]).wait()

        def gather_start(b):
            pltpu.async_copy(feat_hbm.at[sidx.at[b]], rows.at[b], semg[b])

        def gather_wait(b):
            pltpu.make_async_copy(feat_hbm.at[sidx.at[b]], rows.at[b],
                                  semg[b]).wait()

        def _rows_src(b, nr):
            return rows.at[b] if nr == KC else rows.at[b].at[pl.ds(0, nr)]

        def scatter_start(b):
            pltpu.async_copy(_rows_src(b, KC), acc_sh.at[didx.at[b]],
                             sems[b], add=True)

        def scatter_wait(b):
            pltpu.make_async_copy(_rows_src(b, KC), acc_sh.at[didx.at[b]],
                                  sems[b]).wait()

        for c0 in range(LA + 1):
            sidx_start(c0, c0)
            didx_start(c0, c0)
        for c0 in range(LA):
            sidx_wait(c0, c0)
            gather_start(c0)

        @pl.loop(0, nfull, step=NB)
        def _(c):
            for b in range(NB):
                chunk = c + b
                bg = (b + LA) % NB
                bp = (b + NB - 1) % NB

                @pl.when(chunk + LA < nfull)
                def _():
                    sidx_wait(chunk + LA, bg)
                    gather_start(bg)

                gather_wait(b)
                didx_wait(chunk, b)
                scatter_start(b)

                @pl.when(chunk >= 1)
                def _():
                    scatter_wait(bp)

                @pl.when(chunk + LA + 1 < nfull)
                def _():
                    sidx_start(chunk + LA + 1, bp)
                    didx_start(chunk + LA + 1, bp)

        scatter_wait((nfull - 1) % NB)

        if tail_len:
            toff = nfull * KC
            pltpu.async_copy(ei_hbm.at[pl.ds(e_pad + ebase + toff, tail_len)],
                             dtail.at[0], semd[0]).wait()
            pltpu.async_copy(ei_hbm.at[pl.ds(ebase + toff, tail_len)],
                             sidx.at[0].at[pl.ds(0, tail_len)], semx[0]).wait()
            pltpu.async_copy(
                feat_hbm.at[sidx.at[0].at[pl.ds(0, tail_len)]],
                rows.at[0].at[pl.ds(0, tail_len)], semg[0]).wait()
            pltpu.async_copy(rows.at[0].at[pl.ds(0, tail_len)],
                             acc_sh.at[dtail.at[0]], sems[0], add=True).wait()

        plsc.subcore_barrier()

        pltpu.async_copy(acc_sh.at[pl.ds(base, rps)],
                         part_hbm.at[cid, pl.ds(base, rps)], semb).wait()
        if rtail:
            @pl.when(sid == NS - 1)
            def _():
                pltpu.async_copy(acc_sh.at[pl.ds(rps * NS, rtail)],
                                 part_hbm.at[cid, pl.ds(rps * NS, rtail)],
                                 semb).wait()

    return agg_kernel(feat, eflat)


def _l1(hist, x, w1):
    n, d = x.shape
    h = w1.shape[1]

    def body(hist_b, x_b, w1_b, hs_b, dinv_b):
        deg = jnp.sum(hist_b[...], axis=0)
        dinv = lax.rsqrt(jnp.maximum(deg, 1.0))
        dinv_b[...] = dinv[:, None]
        hs_b[...] = jnp.dot(x_b[...], w1_b[...],
                            preferred_element_type=jnp.float32) * dinv[:, None]

    return pl.pallas_call(
        body,
        out_shape=[
            jax.ShapeDtypeStruct((n, h), jnp.float32),
            jax.ShapeDtypeStruct((n, 1), jnp.float32),
        ],
    )(hist, x, w1)


def _layer2(p, dinv, w2):
    _, n, h = p.shape
    c = w2.shape[1]

    def body(p_b, dinv_b, w2_b, o_b):
        s = (p_b[0] + p_b[1]) * dinv_b[...]
        s = jnp.maximum(s, 0.0)
        o_b[...] = jnp.dot(s, w2_b[...],
                           preferred_element_type=jnp.float32) * dinv_b[...]

    return pl.pallas_call(
        body,
        out_shape=jax.ShapeDtypeStruct((n, c), jnp.float32),
    )(p, dinv, w2)


def _final(q, dinv, c):
    _, n, h = q.shape

    def body(q_b, dinv_b, o_b):
        o_b[...] = (q_b[0, :, :c] + q_b[1, :, :c]) * dinv_b[...]

    return pl.pallas_call(
        body,
        out_shape=jax.ShapeDtypeStruct((n, c), jnp.float32),
    )(q, dinv)


def kernel(x, edge_index, W1, W2):
    n, _ = x.shape
    e = edge_index.shape[1]
    c_w = W2.shape[1]
    eflat = edge_index.reshape(2 * e)

    hist = _hist(eflat, n)
    hs, dinv = _l1(hist, x, W1)
    p = _aggregate(hs, eflat)
    hs2 = _layer2(p, dinv, W2)
    q = _aggregate64(hs2, eflat)
    return _final(q, dinv, c_w)


# speedup vs baseline: 1.0218x; 1.0218x over previous
import dataclasses
import functools

import jax
import jax.numpy as jnp
from jax import lax
from jax.experimental import pallas as pl
from jax.experimental.pallas import tpu as pltpu
from jax.experimental.pallas import tpu_sc as plsc

NC = 2
NS = 16
NW = NC * NS
LANES = 16
K = 128
KC = 96
RING = 4
KC64 = 96
RING64 = 4

_MESH = plsc.VectorSubcoreMesh(
    core_axis_name="c", subcore_axis_name="s", num_cores=NC, num_subcores=NS
)

_SC_PARAMS = pltpu.CompilerParams()
if "needs_layout_passes" in pltpu.CompilerParams.__dataclass_fields__:
    _SC_PARAMS = dataclasses.replace(_SC_PARAMS, needs_layout_passes=False)


def _wid():
    return lax.axis_index("s") * NC + lax.axis_index("c")


def _hist(eflat, n_nodes):
    e_pad = eflat.shape[0] // 2
    assert e_pad % NW == 0
    epw = e_pad // NW
    assert epw % LANES == 0

    @functools.partial(
        pl.kernel,
        out_type=jax.ShapeDtypeStruct((NW, n_nodes), jnp.float32),
        mesh=_MESH,
        scratch_types=[
            pltpu.VMEM((n_nodes,), jnp.float32),
            pltpu.VMEM((epw,), jnp.int32),
            pltpu.SemaphoreType.DMA,
        ],
        compiler_params=_SC_PARAMS,
    )
    def hist_kernel(ei_hbm, out_hbm, hist_v, idx_v, sem):
        wid = _wid()
        zero16 = jnp.zeros((LANES,), jnp.float32)
        ones16 = jnp.ones((LANES,), jnp.float32)

        cp = pltpu.async_copy(ei_hbm.at[pl.ds(e_pad + wid * epw, epw)],
                              idx_v, sem)

        @pl.loop(0, n_nodes, step=LANES)
        def _(i):
            hist_v[pl.ds(i, LANES)] = zero16

        cp.wait()

        full = epw // K * K

        @pl.loop(0, full, step=K)
        def _(i):
            for j in range(K // LANES):
                idx16 = idx_v[pl.ds(i + j * LANES, LANES)]
                plsc.addupdate_scatter(hist_v, [idx16], ones16)

        for j in range((epw - full) // LANES):
            idx16 = idx_v[pl.ds(full + j * LANES, LANES)]
            plsc.addupdate_scatter(hist_v, [idx16], ones16)

        pltpu.async_copy(hist_v, out_hbm.at[wid], sem).wait()

    return hist_kernel(eflat)


def _aggregate(feat, eflat):
    n, w = feat.shape
    e_pad = eflat.shape[0] // 2
    assert e_pad % NW == 0
    epw = e_pad // NW
    assert epw % 8 == 0
    nfull = epw // KC
    tail_len = epw - nfull * KC
    NB = RING
    LA = NB - 2
    assert nfull % NB == 0 and nfull > NB, (nfull, NB)
    rps = (n // NS) // 8 * 8
    rtail = n - rps * NS
    assert rtail % 8 == 0

    @functools.partial(
        pl.kernel,
        out_type=jax.ShapeDtypeStruct((NC, n, w), jnp.float32),
        mesh=_MESH,
        scratch_types=[
            pltpu.VMEM_SHARED((n, w), jnp.float32),
            pltpu.VMEM((NB, KC), jnp.int32),
            pltpu.VMEM((NB, KC), jnp.int32),
            pltpu.VMEM((NB, KC, w), jnp.float32),
            pltpu.VMEM((1, max(tail_len, 8)), jnp.int32),
        ] + [pltpu.SemaphoreType.DMA] * (1 + 4 * NB),
        compiler_params=dataclasses.replace(
            _SC_PARAMS, use_tc_tiling_on_sc=False),
    )
    def agg_kernel(feat_hbm, ei_hbm, part_hbm,
                   acc_sh, sidx, didx, rows, dtail, semb, *sems_flat):
        cid = lax.axis_index("c")
        sid = lax.axis_index("s")
        wid = _wid()
        base = sid * rps
        ebase = wid * epw
        semx = sems_flat[0:NB]
        semd = sems_flat[NB:2 * NB]
        semg = sems_flat[2 * NB:3 * NB]
        sems = sems_flat[3 * NB:4 * NB]

        zero16 = jnp.zeros((LANES,), jnp.float32)

        @pl.loop(0, KC)
        def _(r):
            for j in range(w // LANES):
                rows[0, r, pl.ds(j * LANES, LANES)] = zero16

        nz = rps // KC
        zrem = rps - nz * KC

        @pl.loop(0, nz)
        def _(k):
            zoff = pl.multiple_of(base + k * KC, 8)
            pltpu.async_copy(rows.at[0], acc_sh.at[pl.ds(zoff, KC)],
                             semb).wait()
        if zrem:
            pltpu.async_copy(rows.at[0].at[pl.ds(0, zrem)],
                             acc_sh.at[pl.ds(base + nz * KC, zrem)],
                             semb).wait()
        if rtail:
            @pl.when(sid == NS - 1)
            def _():
                pltpu.async_copy(rows.at[0].at[pl.ds(0, rtail)],
                                 acc_sh.at[pl.ds(rps * NS, rtail)],
                                 semb).wait()

        plsc.subcore_barrier()

        def _span(chunk, half):
            return pl.ds(half * e_pad + ebase + chunk * KC, KC)

        def sidx_start(chunk, b):
            pltpu.async_copy(ei_hbm.at[_span(chunk, 0)], sidx.at[b], semx[b])

        def sidx_wait(chunk, b):
            pltpu.make_async_copy(ei_hbm.at[_span(chunk, 0)], sidx.at[b],
                                  semx[b]).wait()

        def didx_start(chunk, b):
            pltpu.async_copy(ei_hbm.at[_span(chunk, 1)], didx.at[b], semd[b])

        def didx_wait(chunk, b):
            pltpu.make_async_copy(ei_hbm.at[_span(chunk, 1)], didx.at[b],
                                  semd[b]).wait()

        def gather_start(b):
            pltpu.async_copy(feat_hbm.at[sidx.at[b]], rows.at[b], semg[b])

        def gather_wait(b):
            pltpu.make_async_copy(feat_hbm.at[sidx.at[b]], rows.at[b],
                                  semg[b]).wait()

        def _rows_src(b, nr):
            return rows.at[b] if nr == KC else rows.at[b].at[pl.ds(0, nr)]

        def scatter_start(b):
            pltpu.async_copy(_rows_src(b, KC), acc_sh.at[didx.at[b]],
                             sems[b], add=True)

        def scatter_wait(b):
            pltpu.make_async_copy(_rows_src(b, KC), acc_sh.at[didx.at[b]],
                                  sems[b]).wait()

        for c0 in range(LA + 1):
            sidx_start(c0, c0)
            didx_start(c0, c0)
        for c0 in range(LA):
            sidx_wait(c0, c0)
            gather_start(c0)

        @pl.loop(0, nfull, step=NB)
        def _(c):
            for b in range(NB):
                chunk = c + b
                bg = (b + LA) % NB
                bp = (b + NB - 1) % NB

                @pl.when(chunk + LA < nfull)
                def _():
                    sidx_wait(chunk + LA, bg)
                    gather_start(bg)

                gather_wait(b)
                didx_wait(chunk, b)
                scatter_start(b)

                @pl.when(chunk >= 1)
                def _():
                    scatter_wait(bp)

                @pl.when(chunk + LA + 1 < nfull)
                def _():
                    sidx_start(chunk + LA + 1, bp)
                    didx_start(chunk + LA + 1, bp)

        scatter_wait((nfull - 1) % NB)

        if tail_len:
            toff = nfull * KC
            pltpu.async_copy(ei_hbm.at[pl.ds(e_pad + ebase + toff, tail_len)],
                             dtail.at[0], semd[0]).wait()
            pltpu.async_copy(ei_hbm.at[pl.ds(ebase + toff, tail_len)],
                             sidx.at[0].at[pl.ds(0, tail_len)], semx[0]).wait()
            pltpu.async_copy(
                feat_hbm.at[sidx.at[0].at[pl.ds(0, tail_len)]],
                rows.at[0].at[pl.ds(0, tail_len)], semg[0]).wait()
            pltpu.async_copy(_rows_src(0, tail_len),
                             acc_sh.at[dtail.at[0]], sems[0], add=True).wait()

        plsc.subcore_barrier()

        pltpu.async_copy(acc_sh.at[pl.ds(base, rps)],
                         part_hbm.at[cid, pl.ds(base, rps)], semb).wait()
        if rtail:
            @pl.when(sid == NS - 1)
            def _():
                pltpu.async_copy(acc_sh.at[pl.ds(rps * NS, rtail)],
                                 part_hbm.at[cid, pl.ds(rps * NS, rtail)],
                                 semb).wait()

    return agg_kernel(feat, eflat)


def _aggregate64(feat, eflat):
    KC = KC64
    n, w = feat.shape
    e_pad = eflat.shape[0] // 2
    assert e_pad % NW == 0
    epw = e_pad // NW
    assert epw % 8 == 0
    nfull = epw // KC
    tail_len = epw - nfull * KC
    NB = RING64
    LA = NB - 2
    assert nfull % NB == 0 and nfull > NB, (nfull, NB)
    rps = (n // NS) // 8 * 8
    rtail = n - rps * NS
    assert rtail % 8 == 0

    @functools.partial(
        pl.kernel,
        out_type=jax.ShapeDtypeStruct((NC, n, w), jnp.float32),
        mesh=_MESH,
        scratch_types=[
            pltpu.VMEM_SHARED((n, w), jnp.float32),
            pltpu.VMEM((NB, KC), jnp.int32),
            pltpu.VMEM((NB, KC), jnp.int32),
            pltpu.VMEM((NB, KC, w), jnp.float32),
            pltpu.VMEM((1, max(tail_len, 8)), jnp.int32),
        ] + [pltpu.SemaphoreType.DMA] * (1 + 4 * NB),
        compiler_params=dataclasses.replace(
            _SC_PARAMS, use_tc_tiling_on_sc=False),
    )
    def agg_kernel(feat_hbm, ei_hbm, part_hbm,
                   acc_sh, sidx, didx, rows, dtail, semb, *sems_flat):
        cid = lax.axis_index("c")
        sid = lax.axis_index("s")
        wid = _wid()
        base = sid * rps
        ebase = wid * epw
        semx = sems_flat[0:NB]
        semd = sems_flat[NB:2 * NB]
        semg = sems_flat[2 * NB:3 * NB]
        sems = sems_flat[3 * NB:4 * NB]

        zero16 = jnp.zeros((LANES,), jnp.float32)

        @pl.loop(0, KC)
        def _(r):
            for j in range(w // LANES):
                rows[0, r, pl.ds(j * LANES, LANES)] = zero16

        nz = rps // KC
        zrem = rps - nz * KC

        @pl.loop(0, nz)
        def _(k):
            zoff = pl.multiple_of(base + k * KC, 8)
            pltpu.async_copy(rows.at[0], acc_sh.at[pl.ds(zoff, KC)],
                             semb).wait()
        if zrem:
            pltpu.async_copy(rows.at[0].at[pl.ds(0, zrem)],
                             acc_sh.at[pl.ds(base + nz * KC, zrem)],
                             semb).wait()
        if rtail:
            @pl.when(sid == NS - 1)
            def _():
                pltpu.async_copy(rows.at[0].at[pl.ds(0, rtail)],
                                 acc_sh.at[pl.ds(rps * NS, rtail)],
                                 semb).wait()

        plsc.subcore_barrier()

        def _span(chunk, half):
            return pl.ds(half * e_pad + ebase + chunk * KC, KC)

        def sidx_start(chunk, b):
            pltpu.async_copy(ei_hbm.at[_span(chunk, 0)], sidx.at[b], semx[b])

        def sidx_wait(chunk, b):
            pltpu.make_async_copy(ei_hbm.at[_span(chunk, 0)], sidx.at[b],
                                  semx[b]).wait()

        def didx_start(chunk, b):
            pltpu.async_copy(ei_hbm.at[_span(chunk, 1)], didx.at[b], semd[b])

        def didx_wait(chunk, b):
            pltpu.make_async_copy(ei_hbm.at[_span(chunk, 1)], didx.at[b],
                                  semd[b]).wait()

        def gather_start(b):
            pltpu.async_copy(feat_hbm.at[sidx.at[b]], rows.at[b], semg[b])

        def gather_wait(b):
            pltpu.make_async_copy(feat_hbm.at[sidx.at[b]], rows.at[b],
                                  semg[b]).wait()

        def _rows_src(b, nr):
            return rows.at[b] if nr == KC else rows.at[b].at[pl.ds(0, nr)]

        def scatter_start(b):
            pltpu.async_copy(_rows_src(b, KC), acc_sh.at[didx.at[b]],
                             sems[b], add=True)

        def scatter_wait(b):
            pltpu.make_async_copy(_rows_src(b, KC), acc_sh.at[didx.at[b]],
                                  sems[b]).wait()

        for c0 in range(LA + 1):
            sidx_start(c0, c0)
            didx_start(c0, c0)
        for c0 in range(LA):
            sidx_wait(c0, c0)
            gather_start(c0)

        @pl.loop(0, nfull, step=NB)
        def _(c):
            for b in range(NB):
                chunk = c + b
                bg = (b + LA) % NB
                bp = (b + NB - 1) % NB

                @pl.when(chunk + LA < nfull)
                def _():
                    sidx_wait(chunk + LA, bg)
                    gather_start(bg)

                gather_wait(b)
                didx_wait(chunk, b)
                scatter_start(b)

                @pl.when(chunk >= 1)
                def _():
                    scatter_wait(bp)

                @pl.when(chunk + LA + 1 < nfull)
                def _():
                    sidx_start(chunk + LA + 1, bp)
                    didx_start(chunk + LA + 1, bp)

        scatter_wait((nfull - 1) % NB)

        if tail_len:
            toff = nfull * KC
            pltpu.async_copy(ei_hbm.at[pl.ds(e_pad + ebase + toff, tail_len)],
                             dtail.at[0], semd[0]).wait()
            pltpu.async_copy(ei_hbm.at[pl.ds(ebase + toff, tail_len)],
                             sidx.at[0].at[pl.ds(0, tail_len)], semx[0]).wait()
            pltpu.async_copy(
                feat_hbm.at[sidx.at[0].at[pl.ds(0, tail_len)]],
                rows.at[0].at[pl.ds(0, tail_len)], semg[0]).wait()
            pltpu.async_copy(rows.at[0].at[pl.ds(0, tail_len)],
                             acc_sh.at[dtail.at[0]], sems[0], add=True).wait()

        plsc.subcore_barrier()

        pltpu.async_copy(acc_sh.at[pl.ds(base, rps)],
                         part_hbm.at[cid, pl.ds(base, rps)], semb).wait()
        if rtail:
            @pl.when(sid == NS - 1)
            def _():
                pltpu.async_copy(acc_sh.at[pl.ds(rps * NS, rtail)],
                                 part_hbm.at[cid, pl.ds(rps * NS, rtail)],
                                 semb).wait()

    return agg_kernel(feat, eflat)


def _l1(hist, x, w1):
    n, d = x.shape
    h = w1.shape[1]

    def body(hist_b, x_b, w1_b, hs_b, dinv_b):
        deg = jnp.sum(hist_b[...], axis=0)
        dinv = lax.rsqrt(jnp.maximum(deg, 1.0))
        dinv_b[...] = dinv[:, None]
        hs_b[...] = jnp.dot(x_b[...], w1_b[...],
                            preferred_element_type=jnp.float32) * dinv[:, None]

    return pl.pallas_call(
        body,
        out_shape=[
            jax.ShapeDtypeStruct((n, h), jnp.float32),
            jax.ShapeDtypeStruct((n, 1), jnp.float32),
        ],
    )(hist, x, w1)


def _layer2(p, dinv, w2):
    _, n, h = p.shape
    c = w2.shape[1]

    def body(p_b, dinv_b, w2_b, o_b):
        s = (p_b[0] + p_b[1]) * dinv_b[...]
        s = jnp.maximum(s, 0.0)
        o_b[...] = jnp.dot(s, w2_b[...],
                           preferred_element_type=jnp.float32) * dinv_b[...]

    return pl.pallas_call(
        body,
        out_shape=jax.ShapeDtypeStruct((n, c), jnp.float32),
    )(p, dinv, w2)


def _final(q, dinv, c):
    _, n, h = q.shape

    def body(q_b, dinv_b, o_b):
        o_b[...] = (q_b[0, :, :c] + q_b[1, :, :c]) * dinv_b[...]

    return pl.pallas_call(
        body,
        out_shape=jax.ShapeDtypeStruct((n, c), jnp.float32),
    )(q, dinv)


def kernel(x, edge_index, W1, W2):
    n, _ = x.shape
    e = edge_index.shape[1]
    c_w = W2.shape[1]
    eflat = edge_index.reshape(2 * e)

    hist = _hist(eflat, n)
    hs, dinv = _l1(hist, x, W1)
    p = _aggregate(hs, eflat)
    hs2 = _layer2(p, dinv, W2)
    q = _aggregate64(hs2, eflat)
    return _final(q, dinv, c_w)
